# baseline (speedup 1.0000x reference)
import functools

import jax
import jax.numpy as jnp
from jax import lax
from jax.experimental import pallas as pl
from jax.experimental.pallas import tpu as pltpu
from jax.experimental.pallas import tpu_sc as plsc

NC = 2
NS = 16
NW = NC * NS
BATCH = 4096
HIST = 50
EMBED = 64
VOCAB = 100000
EPW = EMBED // NW
LANES = 16

_mesh = plsc.VectorSubcoreMesh(
    core_axis_name="c", subcore_axis_name="s", num_cores=NC, num_subcores=NS
)


@functools.partial(
    pl.kernel,
    out_type=jax.ShapeDtypeStruct((HIST, EMBED, BATCH), jnp.float32),
    mesh=_mesh,
    scratch_types=[
        pltpu.VMEM((VOCAB,), jnp.float32),
        pltpu.VMEM((BATCH,), jnp.int32),
        pltpu.VMEM((BATCH,), jnp.float32),
    ],
    compiler_params=pltpu.CompilerParams(needs_layout_passes=False),
)
def _embed_lookup(idx_hbm, table_hbm, out_hbm, telem, idx_v, out_v):
    wid = lax.axis_index("s") * NC + lax.axis_index("c")
    e0 = wid * EPW

    for ei in range(EPW):
        e = e0 + ei
        pltpu.sync_copy(table_hbm.at[e], telem)

        @pl.loop(0, HIST)
        def _h_loop(h):
            pltpu.sync_copy(idx_hbm.at[h], idx_v)

            @pl.loop(0, BATCH // LANES, unroll=8)
            def _i_loop(i):
                iv = idx_v[pl.ds(i * LANES, LANES)]
                out_v[pl.ds(i * LANES, LANES)] = plsc.load_gather(telem, [iv])

            pltpu.sync_copy(out_v, out_hbm.at[h, e])


def kernel(input, weight):
    out_t = _embed_lookup(input.T, weight.T)
    return jnp.transpose(out_t, (2, 0, 1))


# speedup vs baseline: 1.9812x; 1.9812x over previous
import functools

import jax
import jax.numpy as jnp
from jax import lax
from jax.experimental import pallas as pl
from jax.experimental.pallas import tpu as pltpu
from jax.experimental.pallas import tpu_sc as plsc

NC = 2
NS = 16
NW = NC * NS
BATCH = 4096
HIST = 50
EMBED = 64
VOCAB = 100000
EPW = EMBED // NW
LANES = 16

_mesh = plsc.VectorSubcoreMesh(
    core_axis_name="c", subcore_axis_name="s", num_cores=NC, num_subcores=NS
)


@functools.partial(
    pl.kernel,
    out_type=jax.ShapeDtypeStruct((HIST, EMBED, BATCH), jnp.float32),
    mesh=_mesh,
    scratch_types=[
        pltpu.VMEM((VOCAB,), jnp.float32),
        pltpu.VMEM((BATCH,), jnp.int32),
        pltpu.VMEM((BATCH,), jnp.float32),
    ],
    compiler_params=pltpu.CompilerParams(needs_layout_passes=False),
)
def _embed_lookup(idx_hbm, table_hbm, out_hbm, telem, idx_v, out_v):
    wid = lax.axis_index("s") * NC + lax.axis_index("c")
    e0 = wid * EPW

    for ei in range(EPW):
        e = e0 + ei
        pltpu.sync_copy(table_hbm.at[e], telem)

        @pl.loop(0, HIST)
        def _h_loop(h):
            pltpu.sync_copy(idx_hbm.at[h], idx_v)

            @plsc.parallel_loop(0, BATCH // LANES, unroll=8)
            def _i_loop(i):
                iv = idx_v[pl.ds(i * LANES, LANES)]
                out_v[pl.ds(i * LANES, LANES)] = plsc.load_gather(telem, [iv])

            pltpu.sync_copy(out_v, out_hbm.at[h, e])


def kernel(input, weight):
    out_t = _embed_lookup(input.T, weight.T)
    return jnp.transpose(out_t, (2, 0, 1))


# speedup vs baseline: 3.4767x; 1.7548x over previous
import functools

import jax
import jax.numpy as jnp
from jax import lax
from jax.experimental import pallas as pl
from jax.experimental.pallas import tpu as pltpu
from jax.experimental.pallas import tpu_sc as plsc

NC = 2
NS = 16
NW = NC * NS
BATCH = 4096
HIST = 50
EMBED = 64
VOCAB = 100000
EPW = EMBED // NW
LANES = 16

_mesh = plsc.VectorSubcoreMesh(
    core_axis_name="c", subcore_axis_name="s", num_cores=NC, num_subcores=NS
)


@functools.partial(
    pl.kernel,
    out_type=jax.ShapeDtypeStruct((HIST, EMBED, BATCH), jnp.float32),
    mesh=_mesh,
    scratch_types=[
        pltpu.VMEM((VOCAB,), jnp.float32),
        pltpu.VMEM((BATCH,), jnp.int32),
        pltpu.VMEM((BATCH,), jnp.int32),
        pltpu.VMEM((BATCH,), jnp.float32),
        pltpu.VMEM((BATCH,), jnp.float32),
        pltpu.SemaphoreType.DMA,
        pltpu.SemaphoreType.DMA,
        pltpu.SemaphoreType.DMA,
        pltpu.SemaphoreType.DMA,
    ],
    compiler_params=pltpu.CompilerParams(needs_layout_passes=False),
)
def _embed_lookup(idx_hbm, table_hbm, out_hbm, telem, i0, i1, o0, o1, si0, si1, so0, so1):
    idx_b = (i0, i1)
    out_b = (o0, o1)
    isem = (si0, si1)
    osem = (so0, so1)
    wid = lax.axis_index("s") * NC + lax.axis_index("c")
    e0 = wid * EPW

    def fire_idx(h, p):
        pltpu.async_copy(idx_hbm.at[h], idx_b[p], isem[p])

    def wait_idx(h, p):
        pltpu.make_async_copy(idx_hbm.at[h], idx_b[p], isem[p]).wait()

    def wait_store(h, e, p):
        pltpu.make_async_copy(out_b[p], out_hbm.at[h, e], osem[p]).wait()

    for ei in range(EPW):
        e = e0 + ei
        pltpu.sync_copy(table_hbm.at[e], telem)
        fire_idx(0, 0)
        fire_idx(1, 1)

        @pl.loop(0, HIST, step=2)
        def _h_loop(h0):
            for p in range(2):
                h = h0 + p
                wait_idx(h, p)

                @pl.when(h + 2 < HIST)
                def _():
                    fire_idx(h + 2, p)

                @pl.when(h >= 2)
                def _():
                    wait_store(h - 2, e, p)

                @plsc.parallel_loop(0, BATCH // LANES, unroll=8)
                def _i_loop(i):
                    iv = idx_b[p][pl.ds(i * LANES, LANES)]
                    out_b[p][pl.ds(i * LANES, LANES)] = plsc.load_gather(
                        telem, [iv]
                    )

                pltpu.async_copy(out_b[p], out_hbm.at[h, e], osem[p])

        for p in range(2):
            wait_store(HIST - 2 + p, e, p)


def kernel(input, weight):
    out_t = _embed_lookup(input.T, weight.T)
    return jnp.transpose(out_t, (2, 0, 1))


# speedup vs baseline: 3.4783x; 1.0005x over previous
import functools

import jax
import jax.numpy as jnp
from jax import lax
from jax.experimental import pallas as pl
from jax.experimental.pallas import tpu as pltpu
from jax.experimental.pallas import tpu_sc as plsc

NC = 2
NS = 16
NW = NC * NS
BATCH = 4096
HIST = 50
EMBED = 64
VOCAB = 100000
EPW = EMBED // NW
LANES = 16

_mesh = plsc.VectorSubcoreMesh(
    core_axis_name="c", subcore_axis_name="s", num_cores=NC, num_subcores=NS
)


@functools.partial(
    pl.kernel,
    out_type=jax.ShapeDtypeStruct((HIST, EMBED, BATCH), jnp.float32),
    mesh=_mesh,
    scratch_types=[
        pltpu.VMEM((VOCAB,), jnp.float32),
        pltpu.VMEM((BATCH,), jnp.int32),
        pltpu.VMEM((BATCH,), jnp.int32),
        pltpu.VMEM((BATCH,), jnp.float32),
        pltpu.VMEM((BATCH,), jnp.float32),
        pltpu.SemaphoreType.DMA,
        pltpu.SemaphoreType.DMA,
        pltpu.SemaphoreType.DMA,
        pltpu.SemaphoreType.DMA,
    ],
    compiler_params=pltpu.CompilerParams(needs_layout_passes=False),
)
def _embed_lookup(idx_hbm, table_hbm, out_hbm, telem, i0, i1, o0, o1, si0, si1, so0, so1):
    idx_b = (i0, i1)
    out_b = (o0, o1)
    isem = (si0, si1)
    osem = (so0, so1)
    wid = lax.axis_index("s") * NC + lax.axis_index("c")
    e0 = wid * EPW

    def fire_idx(h, p):
        pltpu.async_copy(idx_hbm.at[h], idx_b[p], isem[p])

    def wait_idx(h, p):
        pltpu.make_async_copy(idx_hbm.at[h], idx_b[p], isem[p]).wait()

    def wait_store(h, e, p):
        pltpu.make_async_copy(out_b[p], out_hbm.at[h, e], osem[p]).wait()

    for ei in range(EPW):
        e = e0 + ei
        pltpu.sync_copy(table_hbm.at[e], telem)
        fire_idx(0, 0)
        fire_idx(1, 1)

        @pl.loop(0, HIST, step=2)
        def _h_loop(h0):
            for p in range(2):
                h = h0 + p
                wait_idx(h, p)

                @pl.when(h + 2 < HIST)
                def _():
                    fire_idx(h + 2, p)

                @pl.when(h >= 2)
                def _():
                    wait_store(h - 2, e, p)

                @plsc.parallel_loop(0, BATCH // LANES, unroll=16)
                def _i_loop(i):
                    iv = idx_b[p][pl.ds(i * LANES, LANES)]
                    out_b[p][pl.ds(i * LANES, LANES)] = plsc.load_gather(
                        telem, [iv]
                    )

                pltpu.async_copy(out_b[p], out_hbm.at[h, e], osem[p])

        for p in range(2):
            wait_store(HIST - 2 + p, e, p)


def kernel(input, weight):
    out_t = _embed_lookup(input.T, weight.T)
    return jnp.transpose(out_t, (2, 0, 1))
